# Initial kernel scaffold; baseline (speedup 1.0000x reference)
import functools

import jax
import jax.numpy as jnp
from jax import lax
from jax.experimental import pallas as pl
from jax.experimental.pallas import tpu as pltpu
from jax.experimental.pallas import tpu_sc as plsc

LANES = 16


def _make_sc_kernel(B, L, V, H, C):
    info = plsc.get_sparse_core_info()
    NC, NS = info.num_cores, info.num_subcores
    NW = NC * NS
    samples_per_w = B // NW
    n_chunks = samples_per_w // C
    toks = C * L
    HV = H // LANES

    mesh = plsc.VectorSubcoreMesh(core_axis_name="c", subcore_axis_name="s")

    @functools.partial(
        pl.kernel,
        mesh=mesh,
        out_type=jax.ShapeDtypeStruct((B, H), jnp.float32),
        scratch_types=[
            pltpu.VMEM((toks,), jnp.int32),
            pltpu.VMEM((toks, H), jnp.float32),
            pltpu.VMEM((H,), jnp.float32),
            pltpu.VMEM((LANES,), jnp.float32),
            pltpu.VMEM((C, H), jnp.float32),
            pltpu.SemaphoreType.DMA,
        ],
    )
    def k(x_ref, table_ref, w_ref, b_ref, out_ref, idx_v, emb_v, w_v, b_v, out_v, sem):
        cid = lax.axis_index("c")
        sid = lax.axis_index("s")
        wid = sid * NC + cid
        pltpu.sync_copy(w_ref, w_v)
        pltpu.sync_copy(b_ref, b_v)
        sample0 = wid * samples_per_w

        def chunk_body(i, carry):
            tok_base = (sample0 + i * C) * L
            pltpu.sync_copy(x_ref.at[pl.ds(tok_base, toks)], idx_v)
            pltpu.async_copy(table_ref.at[idx_v], emb_v, sem).wait()

            wvecs = [w_v[pl.ds(j * LANES, LANES)] for j in range(HV)]
            bvec = b_v[...]

            def sample_body(s, carry2):
                row0 = s * L
                acc = [jnp.zeros((LANES,), jnp.float32) for _ in range(HV)]
                dacc = jnp.zeros((LANES,), jnp.float32)
                for l in range(L):
                    r = [emb_v[row0 + l, pl.ds(j * LANES, LANES)] for j in range(HV)]
                    p = r[0] * wvecs[0]
                    for j in range(1, HV):
                        p = p + r[j] * wvecs[j]
                    z = jnp.sum(p)
                    zv = jnp.broadcast_to(z, (LANES,)) + bvec
                    e2 = jnp.exp(zv + zv)
                    t = 1.0 - 2.0 / (e2 + 1.0)
                    u = jnp.exp(t)
                    for j in range(HV):
                        acc[j] = acc[j] + u * r[j]
                    dacc = dacc + u
                inv = 1.0 / dacc
                for j in range(HV):
                    out_v[s, pl.ds(j * LANES, LANES)] = acc[j] * inv
                return carry2

            lax.fori_loop(0, C, sample_body, 0)
            pltpu.sync_copy(out_v, out_ref.at[pl.ds(sample0 + i * C, C)])
            return carry

        lax.fori_loop(0, n_chunks, chunk_body, 0)

    return k


def kernel(x, table, W, b):
    B, L = x.shape
    V, H = table.shape
    x_flat = x.reshape(B * L)
    w_flat = W.reshape(H).astype(jnp.float32)
    b_vec = jnp.broadcast_to(b.reshape(()), (LANES,)).astype(jnp.float32)
    sc = _make_sc_kernel(B, L, V, H, C=8)
    return sc(x_flat, table, w_flat, b_vec)


# speedup vs baseline: 1.7208x; 1.7208x over previous
import functools

import jax
import jax.numpy as jnp
from jax import lax
from jax.experimental import pallas as pl
from jax.experimental.pallas import tpu as pltpu
from jax.experimental.pallas import tpu_sc as plsc

LANES = 16

_GDN = lax.GatherDimensionNumbers(
    offset_dims=(), collapsed_slice_dims=(0,), start_index_map=(0,)
)


def _lane_shuffle(v, perm):
    return lax.gather(
        v,
        perm[:, None],
        dimension_numbers=_GDN,
        slice_sizes=(1,),
        mode=lax.GatherScatterMode.PROMISE_IN_BOUNDS,
    )


def _make_sc_kernel(B, L, V, H, C):
    info = plsc.get_sparse_core_info()
    NC, NS = info.num_cores, info.num_subcores
    NW = NC * NS
    samples_per_w = B // NW
    n_chunks = samples_per_w // C
    toks = C * L
    HV = H // LANES

    mesh = plsc.VectorSubcoreMesh(core_axis_name="c", subcore_axis_name="s")

    @functools.partial(
        pl.kernel,
        mesh=mesh,
        out_type=jax.ShapeDtypeStruct((B, H), jnp.float32),
        compiler_params=pltpu.CompilerParams(use_tc_tiling_on_sc=False),
        scratch_types=[
            pltpu.VMEM((toks,), jnp.int32),
            pltpu.VMEM((toks, H), jnp.float32),
            pltpu.VMEM((H,), jnp.float32),
            pltpu.VMEM((LANES,), jnp.float32),
            pltpu.VMEM((C, H), jnp.float32),
            pltpu.SemaphoreType.DMA,
        ],
    )
    def k(x_ref, table_ref, w_ref, b_ref, out_ref, idx_v, emb_v, w_v, b_v, out_v, sem):
        cid = lax.axis_index("c")
        sid = lax.axis_index("s")
        wid = sid * NC + cid
        pltpu.sync_copy(w_ref, w_v)
        pltpu.sync_copy(b_ref, b_v)
        sample0 = wid * samples_per_w

        def chunk_body(i, carry):
            tok_base = (sample0 + i * C) * L
            pltpu.sync_copy(x_ref.at[pl.ds(tok_base, toks)], idx_v)
            pltpu.async_copy(table_ref.at[idx_v], emb_v, sem).wait()

            wvecs = [w_v[pl.ds(j * LANES, LANES)] for j in range(HV)]
            bvec = b_v[...]

            def sample_body(s, carry2):
                row0 = s * L
                acc = [jnp.zeros((LANES,), jnp.float32) for _ in range(HV)]
                dacc = jnp.zeros((LANES,), jnp.float32)
                for l in range(L):
                    r = [emb_v[row0 + l, pl.ds(j * LANES, LANES)] for j in range(HV)]
                    p = r[0] * wvecs[0]
                    for j in range(1, HV):
                        p = p + r[j] * wvecs[j]
                    for k in (8, 4, 2, 1):
                        perm = jnp.bitwise_xor(lax.iota(jnp.int32, LANES), k)
                        p = p + _lane_shuffle(p, perm)
                    zv = p + bvec
                    e2 = jnp.exp(zv + zv)
                    t = 1.0 - 2.0 / (e2 + 1.0)
                    u = jnp.exp(t)
                    for j in range(HV):
                        acc[j] = acc[j] + u * r[j]
                    dacc = dacc + u
                inv = 1.0 / dacc
                for j in range(HV):
                    out_v[s, pl.ds(j * LANES, LANES)] = acc[j] * inv
                return carry2

            lax.fori_loop(0, C, sample_body, 0)
            pltpu.sync_copy(out_v, out_ref.at[pl.ds(sample0 + i * C, C)])
            return carry

        lax.fori_loop(0, n_chunks, chunk_body, 0)

    return k


def kernel(x, table, W, b):
    B, L = x.shape
    V, H = table.shape
    x_flat = x.reshape(B * L)
    w_flat = W.reshape(H).astype(jnp.float32)
    b_vec = jnp.broadcast_to(b.reshape(()), (LANES,)).astype(jnp.float32)
    sc = _make_sc_kernel(B, L, V, H, C=8)
    return sc(x_flat, table, w_flat, b_vec)


# speedup vs baseline: 1.9858x; 1.1540x over previous
import functools

import jax
import jax.numpy as jnp
from jax import lax
from jax.experimental import pallas as pl
from jax.experimental.pallas import tpu as pltpu
from jax.experimental.pallas import tpu_sc as plsc

LANES = 16
LOG2E = 1.4426950408889634


def _make_sc_kernel(B, L, V, H, C):
    info = plsc.get_sparse_core_info()
    NC, NS = info.num_cores, info.num_subcores
    NW = NC * NS
    samples_per_w = B // NW
    n_chunks = samples_per_w // C
    toks = C * L
    HV = H // LANES
    n_groups = toks // LANES
    assert toks % LANES == 0 and n_chunks % 2 == 0

    mesh = plsc.VectorSubcoreMesh(core_axis_name="c", subcore_axis_name="s")

    @functools.partial(
        pl.kernel,
        mesh=mesh,
        out_type=jax.ShapeDtypeStruct((B, H), jnp.float32),
        compiler_params=pltpu.CompilerParams(
            use_tc_tiling_on_sc=False, needs_layout_passes=False
        ),
        scratch_types=[
            pltpu.VMEM((toks,), jnp.int32),
            pltpu.VMEM((toks,), jnp.int32),
            pltpu.VMEM((toks, H), jnp.float32),
            pltpu.VMEM((toks, H), jnp.float32),
            pltpu.VMEM((LANES, toks), jnp.float32),
            pltpu.VMEM((toks + LANES,), jnp.float32),
            pltpu.VMEM((H,), jnp.float32),
            pltpu.VMEM((LANES,), jnp.float32),
            pltpu.VMEM((C, H), jnp.float32),
            pltpu.SemaphoreType.DMA,
            pltpu.SemaphoreType.DMA,
        ],
    )
    def k(x_ref, table_ref, w_ref, b_ref, out_ref,
          idx_a, idx_b, emb_a, emb_b, pbuf, ubuf, w_v, b_v, out_v,
          sem_a, sem_b):
        cid = lax.axis_index("c")
        sid = lax.axis_index("s")
        wid = sid * NC + cid
        pltpu.sync_copy(w_ref, w_v)
        pltpu.sync_copy(b_ref, b_v)
        sample0 = wid * samples_per_w
        rowids = lax.iota(jnp.int32, LANES)

        def compute(emb_v, chunk):
            wvecs = [w_v[pl.ds(j * LANES, LANES)] for j in range(HV)]
            b2 = b_v[...] * 2.0

            def pass_a(blk, _):
                for dl in range(UNROLL_A):
                    t = blk * UNROLL_A + dl
                    p = emb_v[t, pl.ds(0, LANES)] * wvecs[0]
                    for j in range(1, HV):
                        p = p + emb_v[t, pl.ds(j * LANES, LANES)] * wvecs[j]
                    col = jnp.broadcast_to(t, (LANES,)).astype(jnp.int32)
                    plsc.store_scatter(pbuf, [rowids, col], p)
                return _

            UNROLL_A = 10
            lax.fori_loop(0, toks // UNROLL_A, pass_a, 0)

            def pass_b(gb, _):
                for dg in range(UNROLL_B):
                    g = gb * UNROLL_B + dg
                    z = pbuf[0, pl.ds(g * LANES, LANES)]
                    for r in range(1, LANES):
                        z = z + pbuf[r, pl.ds(g * LANES, LANES)]
                    e2 = jnp.exp(z + z + b2)
                    t = 1.0 - 2.0 / (e2 + 1.0)
                    u = jnp.exp(t)
                    ubuf[pl.ds(g * LANES, LANES)] = u
                return _

            UNROLL_B = 5
            lax.fori_loop(0, n_groups // UNROLL_B, pass_b, 0)

            def pass_c(s, _):
                row0 = s * L
                acc = [jnp.zeros((LANES,), jnp.float32) for _ in range(HV)]
                dacc = jnp.zeros((LANES,), jnp.float32)
                for l in range(L):
                    uv = ubuf[pl.ds(row0 + l, LANES)]
                    u = jnp.broadcast_to(uv[0], (LANES,))
                    for j in range(HV):
                        acc[j] = acc[j] + u * emb_v[row0 + l, pl.ds(j * LANES, LANES)]
                    dacc = dacc + u
                inv = 1.0 / dacc
                for j in range(HV):
                    out_v[s, pl.ds(j * LANES, LANES)] = acc[j] * inv
                return _

            lax.fori_loop(0, C, pass_c, 0)
            pltpu.sync_copy(out_v, out_ref.at[pl.ds(sample0 + chunk * C, C)])

        def pair_body(i, _):
            c0 = i * 2
            c1 = c0 + 1
            pltpu.sync_copy(x_ref.at[pl.ds((sample0 + c0 * C) * L, toks)], idx_a)
            ha = pltpu.async_copy(table_ref.at[idx_a], emb_a, sem_a)
            pltpu.sync_copy(x_ref.at[pl.ds((sample0 + c1 * C) * L, toks)], idx_b)
            hb = pltpu.async_copy(table_ref.at[idx_b], emb_b, sem_b)
            ha.wait()
            compute(emb_a, c0)
            hb.wait()
            compute(emb_b, c1)
            return _

        lax.fori_loop(0, n_chunks // 2, pair_body, 0)

    return k


def kernel(x, table, W, b):
    B, L = x.shape
    V, H = table.shape
    x_flat = x.reshape(B * L)
    w_flat = W.reshape(H).astype(jnp.float32)
    b_vec = jnp.broadcast_to(b.reshape(()), (LANES,)).astype(jnp.float32)
    sc = _make_sc_kernel(B, L, V, H, C=8)
    return sc(x_flat, table, w_flat, b_vec)


# speedup vs baseline: 2.5705x; 1.2945x over previous
import functools

import jax
import jax.numpy as jnp
from jax import lax
from jax.experimental import pallas as pl
from jax.experimental.pallas import tpu as pltpu
from jax.experimental.pallas import tpu_sc as plsc

LANES = 16


def _utable_tc(tableT, W, b):
    V = tableT.shape[1]
    H = tableT.shape[0]
    BN = 16384
    grid = (V + BN - 1) // BN

    def body(t_ref, w_ref, b_ref, u_ref):
        blk = t_ref[...]
        wb = jnp.broadcast_to(w_ref[...], (H, BN))
        s = jnp.sum(blk * wb, axis=0) + b_ref[0]
        u_ref[...] = jnp.exp(jnp.tanh(s))

    return pl.pallas_call(
        body,
        grid=grid,
        in_specs=[
            pl.BlockSpec((H, BN), lambda j: (0, j)),
            pl.BlockSpec((H, 1), lambda j: (0, 0)),
            pl.BlockSpec(memory_space=pltpu.SMEM),
        ],
        out_specs=pl.BlockSpec((BN,), lambda j: (j,)),
        out_shape=jax.ShapeDtypeStruct((V,), jnp.float32),
    )(tableT, W, b)


def _make_sc_kernel(B, L, V, H, C):
    info = plsc.get_sparse_core_info()
    NC, NS = info.num_cores, info.num_subcores
    NW = NC * NS
    samples_per_w = B // NW
    n_chunks = samples_per_w // C
    toks = C * L
    HV = H // LANES
    n_ugrp = (L + LANES - 1) // LANES
    assert n_chunks % 2 == 0

    mesh = plsc.VectorSubcoreMesh(core_axis_name="c", subcore_axis_name="s")

    @functools.partial(
        pl.kernel,
        mesh=mesh,
        out_type=jax.ShapeDtypeStruct((B, H), jnp.float32),
        compiler_params=pltpu.CompilerParams(
            use_tc_tiling_on_sc=False, needs_layout_passes=False
        ),
        scratch_types=[
            pltpu.VMEM((toks,), jnp.int32),
            pltpu.VMEM((toks,), jnp.int32),
            pltpu.VMEM((toks, H), jnp.float32),
            pltpu.VMEM((toks, H), jnp.float32),
            pltpu.VMEM((toks + LANES,), jnp.float32),
            pltpu.VMEM((toks + LANES,), jnp.float32),
            pltpu.VMEM((C, H), jnp.float32),
            pltpu.SemaphoreType.DMA,
            pltpu.SemaphoreType.DMA,
            pltpu.SemaphoreType.DMA,
            pltpu.SemaphoreType.DMA,
        ],
    )
    def k(x_ref, table_ref, utab_ref, out_ref,
          idx_a, idx_b, emb_a, emb_b, u_a, u_b, out_v,
          sem_ra, sem_rb, sem_ua, sem_ub):
        cid = lax.axis_index("c")
        sid = lax.axis_index("s")
        wid = sid * NC + cid
        sample0 = wid * samples_per_w

        def compute(emb_v, u_v, chunk):
            def sample_body(s, _):
                row0 = s * L
                uvecs = [u_v[pl.ds(row0 + g * LANES, LANES)] for g in range(n_ugrp)]
                acc = [jnp.zeros((LANES,), jnp.float32) for _ in range(HV)]
                dacc = jnp.zeros((LANES,), jnp.float32)
                for l in range(L):
                    u = jnp.broadcast_to(uvecs[l // LANES][l % LANES], (LANES,))
                    for j in range(HV):
                        acc[j] = acc[j] + u * emb_v[row0 + l, pl.ds(j * LANES, LANES)]
                    dacc = dacc + u
                inv = 1.0 / dacc
                for j in range(HV):
                    out_v[s, pl.ds(j * LANES, LANES)] = acc[j] * inv
                return _

            lax.fori_loop(0, C, sample_body, 0)
            pltpu.sync_copy(out_v, out_ref.at[pl.ds(sample0 + chunk * C, C)])

        def pair_body(i, _):
            c0 = i * 2
            c1 = c0 + 1
            pltpu.sync_copy(x_ref.at[pl.ds((sample0 + c0 * C) * L, toks)], idx_a)
            h_ra = pltpu.async_copy(table_ref.at[idx_a], emb_a, sem_ra)
            h_ua = pltpu.async_copy(utab_ref.at[idx_a], u_a.at[pl.ds(0, toks)], sem_ua)
            pltpu.sync_copy(x_ref.at[pl.ds((sample0 + c1 * C) * L, toks)], idx_b)
            h_rb = pltpu.async_copy(table_ref.at[idx_b], emb_b, sem_rb)
            h_ub = pltpu.async_copy(utab_ref.at[idx_b], u_b.at[pl.ds(0, toks)], sem_ub)
            h_ra.wait()
            h_ua.wait()
            compute(emb_a, u_a, c0)
            h_rb.wait()
            h_ub.wait()
            compute(emb_b, u_b, c1)
            return _

        lax.fori_loop(0, n_chunks // 2, pair_body, 0)

    return k


def kernel(x, table, W, b):
    B, L = x.shape
    V, H = table.shape
    x_flat = x.reshape(B * L)
    utab = _utable_tc(table.T, W, b)
    sc = _make_sc_kernel(B, L, V, H, C=8)
    return sc(x_flat, table, utab)


# speedup vs baseline: 3.8086x; 1.4817x over previous
import functools

import jax
import jax.numpy as jnp
from jax import lax
from jax.experimental import pallas as pl
from jax.experimental.pallas import tpu as pltpu
from jax.experimental.pallas import tpu_sc as plsc

LANES = 16


TC_BN = 32768


def _utable_tc(tableT, W, b):
    V = tableT.shape[1]
    H = tableT.shape[0]
    BN = TC_BN
    grid = (V + BN - 1) // BN
    R = grid * (BN // 2)

    def body(t_ref, w_ref, b_ref, u_ref, p_ref):
        eye = (lax.broadcasted_iota(jnp.int32, (H, H), 0)
               == lax.broadcasted_iota(jnp.int32, (H, H), 1)).astype(jnp.float32)
        blk = t_ref[...]
        wb = jnp.broadcast_to(w_ref[...], (H, BN))
        s = jnp.sum(blk * wb, axis=0) + b_ref[0]
        u_ref[...] = jnp.exp(jnp.tanh(s))
        dn = (((0,), (0,)), ((), ()))
        ta = lax.dot_general(blk[:, : BN // 2], eye, dn,
                             preferred_element_type=jnp.float32)
        tb = lax.dot_general(blk[:, BN // 2:], eye, dn,
                             preferred_element_type=jnp.float32)
        p_ref[:, pl.ds(0, H)] = ta
        p_ref[:, pl.ds(H, H)] = tb

    return pl.pallas_call(
        body,
        grid=grid,
        in_specs=[
            pl.BlockSpec((H, BN), lambda j: (0, j)),
            pl.BlockSpec((H, 1), lambda j: (0, 0)),
            pl.BlockSpec(memory_space=pltpu.SMEM),
        ],
        out_specs=[
            pl.BlockSpec((BN,), lambda j: (j,)),
            pl.BlockSpec((BN // 2, 2 * H), lambda j: (j, 0)),
        ],
        out_shape=[
            jax.ShapeDtypeStruct((V,), jnp.float32),
            jax.ShapeDtypeStruct((R, 2 * H), jnp.float32),
        ],
    )(tableT, W, b)


def _make_sc_kernel(B, L, R, H, C):
    info = plsc.get_sparse_core_info()
    NC, NS = info.num_cores, info.num_subcores
    NW = NC * NS
    samples_per_w = B // NW
    n_chunks = samples_per_w // C
    toks = C * L
    HV = H // LANES
    n_ugrp = (L + LANES - 1) // LANES
    n_grp = toks // LANES
    W2 = 2 * H
    assert n_chunks % 2 == 0 and toks % LANES == 0

    mesh = plsc.VectorSubcoreMesh(core_axis_name="c", subcore_axis_name="s")

    @functools.partial(
        pl.kernel,
        mesh=mesh,
        out_type=jax.ShapeDtypeStruct((B, H), jnp.float32),
        compiler_params=pltpu.CompilerParams(
            use_tc_tiling_on_sc=False, needs_layout_passes=False
        ),
        scratch_types=[
            pltpu.VMEM((toks,), jnp.int32),
            pltpu.VMEM((toks,), jnp.int32),
            pltpu.VMEM((toks,), jnp.int32),
            pltpu.VMEM((toks,), jnp.int32),
            pltpu.VMEM((toks, W2), jnp.float32),
            pltpu.VMEM((toks, W2), jnp.float32),
            pltpu.VMEM((toks,), jnp.float32),
            pltpu.VMEM((toks,), jnp.float32),
            pltpu.VMEM((C, H), jnp.float32),
            pltpu.SemaphoreType.DMA,
            pltpu.SemaphoreType.DMA,
            pltpu.SemaphoreType.DMA,
            pltpu.SemaphoreType.DMA,
        ],
    )
    def k(x_ref, table_ref, utab_ref, out_ref,
          idx_a, idx_b, idx2_a, idx2_b, emb_a, emb_b, u_a, u_b, out_v,
          sem_ra, sem_rb, sem_ua, sem_ub):
        cid = lax.axis_index("c")
        sid = lax.axis_index("s")
        wid = sid * NC + cid
        sample0 = wid * samples_per_w

        def stage(chunk, idx_v, idx2_v, emb_v, u_v, sem_r, sem_u):
            pltpu.sync_copy(x_ref.at[pl.ds((sample0 + chunk * C) * L, toks)],
                            idx_v)
            for g in range(n_grp):
                sl = pl.ds(g * LANES, LANES)
                v = idx_v[sl]
                idx2_v[sl] = jnp.bitwise_or(
                    lax.shift_left(lax.shift_right_logical(v, 15), 14),
                    jnp.bitwise_and(v, (TC_BN // 2) - 1),
                )
            h_r = pltpu.async_copy(table_ref.at[idx2_v], emb_v, sem_r)
            h_u = pltpu.async_copy(utab_ref.at[idx_v], u_v, sem_u)
            return h_r, h_u

        def compute(idx_v, emb_v, u_v, chunk):
            def sample_body(s, _):
                row0 = s * L
                bases = [min(g * LANES, L - LANES) for g in range(n_ugrp)]
                uvecs = [u_v[pl.ds(row0 + bg, LANES)] for bg in bases]
                offv = [
                    lax.shift_left(
                        jnp.bitwise_and(
                            lax.shift_right_logical(
                                idx_v[pl.ds(row0 + bg, LANES)], 14),
                            1,
                        ),
                        6,
                    )
                    for bg in bases
                ]
                acc = [jnp.zeros((LANES,), jnp.float32) for _ in range(HV)]
                dacc = jnp.zeros((LANES,), jnp.float32)
                for l in range(L):
                    g = min(l // LANES, n_ugrp - 1)
                    lane = l - bases[g]
                    u = jnp.broadcast_to(uvecs[g][lane], (LANES,))
                    off = offv[g][lane]
                    for j in range(HV):
                        acc[j] = acc[j] + u * emb_v[row0 + l,
                                                    pl.ds(off + j * LANES, LANES)]
                    dacc = dacc + u
                inv = 1.0 / dacc
                for j in range(HV):
                    out_v[s, pl.ds(j * LANES, LANES)] = acc[j] * inv
                return _

            lax.fori_loop(0, C, sample_body, 0)
            pltpu.sync_copy(out_v, out_ref.at[pl.ds(sample0 + chunk * C, C)])

        def pair_body(i, _):
            c0 = i * 2
            c1 = c0 + 1
            h_ra, h_ua = stage(c0, idx_a, idx2_a, emb_a, u_a, sem_ra, sem_ua)
            h_rb, h_ub = stage(c1, idx_b, idx2_b, emb_b, u_b, sem_rb, sem_ub)
            h_ra.wait()
            h_ua.wait()
            compute(idx_a, emb_a, u_a, c0)
            h_rb.wait()
            h_ub.wait()
            compute(idx_b, emb_b, u_b, c1)
            return _

        lax.fori_loop(0, n_chunks // 2, pair_body, 0)

    return k


def kernel(x, table, W, b):
    B, L = x.shape
    V, H = table.shape
    x_flat = x.reshape(B * L)
    utab, packed = _utable_tc(table.T, W, b)
    sc = _make_sc_kernel(B, L, packed.shape[0], H, C=8)
    return sc(x_flat, packed, utab)


# speedup vs baseline: 4.5905x; 1.2053x over previous
import functools

import jax
import jax.numpy as jnp
from jax import lax
from jax.experimental import pallas as pl
from jax.experimental.pallas import tpu as pltpu
from jax.experimental.pallas import tpu_sc as plsc

LANES = 16


TC_BN = 32768


def _utable_tc(tableT, W, b):
    V = tableT.shape[1]
    H = tableT.shape[0]
    BN = TC_BN
    grid = (V + BN - 1) // BN
    R = grid * (BN // 2)

    def body(t_ref, w_ref, b_ref, u_ref, p_ref):
        eye = (lax.broadcasted_iota(jnp.int32, (H, H), 0)
               == lax.broadcasted_iota(jnp.int32, (H, H), 1)).astype(jnp.float32)
        blk = t_ref[...]
        wb = jnp.broadcast_to(w_ref[...], (H, BN))
        s = jnp.sum(blk * wb, axis=0) + b_ref[0]
        u_ref[...] = jnp.exp(jnp.tanh(s))
        dn = (((0,), (0,)), ((), ()))
        ta = lax.dot_general(blk[:, : BN // 2], eye, dn,
                             preferred_element_type=jnp.float32)
        tb = lax.dot_general(blk[:, BN // 2:], eye, dn,
                             preferred_element_type=jnp.float32)
        p_ref[:, pl.ds(0, H)] = ta
        p_ref[:, pl.ds(H, H)] = tb

    return pl.pallas_call(
        body,
        grid=grid,
        in_specs=[
            pl.BlockSpec((H, BN), lambda j: (0, j)),
            pl.BlockSpec((H, 1), lambda j: (0, 0)),
            pl.BlockSpec(memory_space=pltpu.SMEM),
        ],
        out_specs=[
            pl.BlockSpec((BN,), lambda j: (j,)),
            pl.BlockSpec((BN // 2, 2 * H), lambda j: (j, 0)),
        ],
        out_shape=[
            jax.ShapeDtypeStruct((V,), jnp.float32),
            jax.ShapeDtypeStruct((R, 2 * H), jnp.float32),
        ],
    )(tableT, W, b)


def _make_sc_kernel(B, L, R64, H, C):
    info = plsc.get_sparse_core_info()
    NC, NS = info.num_cores, info.num_subcores
    NW = NC * NS
    samples_per_w = B // NW
    n_chunks = samples_per_w // C
    toks = C * L
    toks2 = 2 * toks
    HV = H // LANES
    n_ugrp = (L + LANES - 1) // LANES
    n_grp2 = toks2 // LANES
    KB = TC_BN // 2
    assert n_chunks % 2 == 0 and toks % LANES == 0

    mesh = plsc.VectorSubcoreMesh(core_axis_name="c", subcore_axis_name="s")

    @functools.partial(
        pl.kernel,
        mesh=mesh,
        out_type=jax.ShapeDtypeStruct((B, H), jnp.float32),
        compiler_params=pltpu.CompilerParams(
            use_tc_tiling_on_sc=False, needs_layout_passes=False
        ),
        scratch_types=[
            pltpu.VMEM((toks2,), jnp.int32),
            pltpu.VMEM((toks2,), jnp.int32),
            pltpu.VMEM((toks, H), jnp.float32),
            pltpu.VMEM((toks, H), jnp.float32),
            pltpu.VMEM((toks,), jnp.float32),
            pltpu.VMEM((toks,), jnp.float32),
            pltpu.VMEM((C, H), jnp.float32),
            pltpu.SemaphoreType.DMA,
            pltpu.SemaphoreType.DMA,
            pltpu.SemaphoreType.DMA,
            pltpu.SemaphoreType.DMA,
        ],
    )
    def k(x_ref, table_ref, utab_ref, out_ref,
          idx_v, idx2_v, emb_a, emb_b, u_a, u_b, out_v,
          sem_ra, sem_rb, sem_ua, sem_ub):
        cid = lax.axis_index("c")
        sid = lax.axis_index("s")
        wid = sid * NC + cid
        sample0 = wid * samples_per_w

        def compute(emb_v, u_v, chunk):
            def sample_body(s, _):
                row0 = s * L
                bases = [min(g * LANES, L - LANES) for g in range(n_ugrp)]
                uvecs = [u_v[pl.ds(row0 + bg, LANES)] for bg in bases]
                acc = [jnp.zeros((LANES,), jnp.float32) for _ in range(HV)]
                dacc = jnp.zeros((LANES,), jnp.float32)
                for l in range(L):
                    g = min(l // LANES, n_ugrp - 1)
                    lane = l - bases[g]
                    u = jnp.broadcast_to(uvecs[g][lane], (LANES,))
                    for j in range(HV):
                        acc[j] = acc[j] + u * emb_v[row0 + l,
                                                    pl.ds(j * LANES, LANES)]
                    dacc = dacc + u
                inv = 1.0 / dacc
                for j in range(HV):
                    out_v[s, pl.ds(j * LANES, LANES)] = acc[j] * inv
                return _

            lax.fori_loop(0, C, sample_body, 0)
            pltpu.sync_copy(out_v, out_ref.at[pl.ds(sample0 + chunk * C, C)])

        def pair_body(i, _):
            c0 = i * 2
            c1 = c0 + 1
            pltpu.sync_copy(
                x_ref.at[pl.ds((sample0 + c0 * C) * L, toks2)], idx_v)
            for g in range(n_grp2):
                sl = pl.ds(g * LANES, LANES)
                v = idx_v[sl]
                hi = lax.shift_left(lax.shift_right_logical(v, 15), 15)
                mid = lax.shift_left(jnp.bitwise_and(v, KB - 1), 1)
                par = jnp.bitwise_and(lax.shift_right_logical(v, 14), 1)
                idx2_v[sl] = jnp.bitwise_or(hi, jnp.bitwise_or(mid, par))
            h_ra = pltpu.async_copy(
                table_ref.at[idx2_v.at[pl.ds(0, toks)]], emb_a, sem_ra)
            h_ua = pltpu.async_copy(
                utab_ref.at[idx_v.at[pl.ds(0, toks)]], u_a, sem_ua)
            h_rb = pltpu.async_copy(
                table_ref.at[idx2_v.at[pl.ds(toks, toks)]], emb_b, sem_rb)
            h_ub = pltpu.async_copy(
                utab_ref.at[idx_v.at[pl.ds(toks, toks)]], u_b, sem_ub)
            h_ra.wait()
            h_ua.wait()
            compute(emb_a, u_a, c0)
            h_rb.wait()
            h_ub.wait()
            compute(emb_b, u_b, c1)
            return _

        lax.fori_loop(0, n_chunks // 2, pair_body, 0)

    return k


def kernel(x, table, W, b):
    B, L = x.shape
    V, H = table.shape
    x_flat = x.reshape(B * L)
    utab, packed = _utable_tc(table.T, W, b)
    packed64 = packed.reshape(packed.shape[0] * 2, H)
    sc = _make_sc_kernel(B, L, packed64.shape[0], H, C=16)
    return sc(x_flat, packed64, utab)


# speedup vs baseline: 4.6071x; 1.0036x over previous
import functools

import jax
import jax.numpy as jnp
from jax import lax
from jax.experimental import pallas as pl
from jax.experimental.pallas import tpu as pltpu
from jax.experimental.pallas import tpu_sc as plsc

LANES = 16


TC_BN = 32768


def _utable_tc(tableT, W, b):
    V = tableT.shape[1]
    H = tableT.shape[0]
    BN = TC_BN
    grid = (V + BN - 1) // BN
    R = grid * (BN // 2)

    def body(t_ref, w_ref, b_ref, u_ref, p_ref):
        blk = t_ref[...]
        wb = jnp.broadcast_to(w_ref[...], (H, BN))
        s = jnp.sum(blk * wb, axis=0) + b_ref[0]
        u_ref[...] = jnp.exp(jnp.tanh(s))
        ta = jnp.swapaxes(blk[:, : BN // 2], 0, 1)
        tb = jnp.swapaxes(blk[:, BN // 2:], 0, 1)
        p_ref[:, pl.ds(0, H)] = ta
        p_ref[:, pl.ds(H, H)] = tb

    return pl.pallas_call(
        body,
        grid=grid,
        in_specs=[
            pl.BlockSpec((H, BN), lambda j: (0, j)),
            pl.BlockSpec((H, 1), lambda j: (0, 0)),
            pl.BlockSpec(memory_space=pltpu.SMEM),
        ],
        out_specs=[
            pl.BlockSpec((BN,), lambda j: (j,)),
            pl.BlockSpec((BN // 2, 2 * H), lambda j: (j, 0)),
        ],
        out_shape=[
            jax.ShapeDtypeStruct((V,), jnp.float32),
            jax.ShapeDtypeStruct((R, 2 * H), jnp.float32),
        ],
    )(tableT, W, b)


def _make_sc_kernel(B, L, R64, H, C):
    info = plsc.get_sparse_core_info()
    NC, NS = info.num_cores, info.num_subcores
    NW = NC * NS
    samples_per_w = B // NW
    n_chunks = samples_per_w // C
    toks = C * L
    toks2 = 2 * toks
    HV = H // LANES
    n_ugrp = (L + LANES - 1) // LANES
    n_grp2 = toks2 // LANES
    KB = TC_BN // 2
    assert n_chunks % 2 == 0 and toks % LANES == 0

    mesh = plsc.VectorSubcoreMesh(core_axis_name="c", subcore_axis_name="s")

    @functools.partial(
        pl.kernel,
        mesh=mesh,
        out_type=jax.ShapeDtypeStruct((B, H), jnp.float32),
        compiler_params=pltpu.CompilerParams(
            use_tc_tiling_on_sc=False, needs_layout_passes=False
        ),
        scratch_types=[
            pltpu.VMEM((toks2,), jnp.int32),
            pltpu.VMEM((toks2,), jnp.int32),
            pltpu.VMEM((toks, H), jnp.float32),
            pltpu.VMEM((toks, H), jnp.float32),
            pltpu.VMEM((toks,), jnp.float32),
            pltpu.VMEM((toks,), jnp.float32),
            pltpu.VMEM((C, H), jnp.float32),
            pltpu.SemaphoreType.DMA,
            pltpu.SemaphoreType.DMA,
            pltpu.SemaphoreType.DMA,
            pltpu.SemaphoreType.DMA,
        ],
    )
    def k(x_ref, table_ref, utab_ref, out_ref,
          idx_v, idx2_v, emb_a, emb_b, u_a, u_b, out_v,
          sem_ra, sem_rb, sem_ua, sem_ub):
        cid = lax.axis_index("c")
        sid = lax.axis_index("s")
        wid = sid * NC + cid
        sample0 = wid * samples_per_w

        def compute(emb_v, u_v, chunk):
            def sample_body(s, _):
                row0 = s * L
                bases = [min(g * LANES, L - LANES) for g in range(n_ugrp)]
                uvecs = [u_v[pl.ds(row0 + bg, LANES)] for bg in bases]
                acc = [jnp.zeros((LANES,), jnp.float32) for _ in range(HV)]
                dacc = jnp.zeros((LANES,), jnp.float32)
                for l in range(L):
                    g = min(l // LANES, n_ugrp - 1)
                    lane = l - bases[g]
                    u = jnp.broadcast_to(uvecs[g][lane], (LANES,))
                    for j in range(HV):
                        acc[j] = acc[j] + u * emb_v[row0 + l,
                                                    pl.ds(j * LANES, LANES)]
                    dacc = dacc + u
                inv = 1.0 / dacc
                for j in range(HV):
                    out_v[s, pl.ds(j * LANES, LANES)] = acc[j] * inv
                return _

            lax.fori_loop(0, C, sample_body, 0)
            pltpu.sync_copy(out_v, out_ref.at[pl.ds(sample0 + chunk * C, C)])

        def pair_body(i, _):
            c0 = i * 2
            c1 = c0 + 1
            pltpu.sync_copy(
                x_ref.at[pl.ds((sample0 + c0 * C) * L, toks2)], idx_v)
            for g in range(n_grp2):
                sl = pl.ds(g * LANES, LANES)
                v = idx_v[sl]
                hi = lax.shift_left(lax.shift_right_logical(v, 15), 15)
                mid = lax.shift_left(jnp.bitwise_and(v, KB - 1), 1)
                par = jnp.bitwise_and(lax.shift_right_logical(v, 14), 1)
                idx2_v[sl] = jnp.bitwise_or(hi, jnp.bitwise_or(mid, par))
            h_ra = pltpu.async_copy(
                table_ref.at[idx2_v.at[pl.ds(0, toks)]], emb_a, sem_ra)
            h_ua = pltpu.async_copy(
                utab_ref.at[idx_v.at[pl.ds(0, toks)]], u_a, sem_ua)
            h_rb = pltpu.async_copy(
                table_ref.at[idx2_v.at[pl.ds(toks, toks)]], emb_b, sem_rb)
            h_ub = pltpu.async_copy(
                utab_ref.at[idx_v.at[pl.ds(toks, toks)]], u_b, sem_ub)
            h_ra.wait()
            h_ua.wait()
            compute(emb_a, u_a, c0)
            h_rb.wait()
            h_ub.wait()
            compute(emb_b, u_b, c1)
            return _

        lax.fori_loop(0, n_chunks // 2, pair_body, 0)

    return k


def kernel(x, table, W, b):
    B, L = x.shape
    V, H = table.shape
    x_flat = x.reshape(B * L)
    utab, packed = _utable_tc(table.T, W, b)
    packed64 = packed.reshape(packed.shape[0] * 2, H)
    sc = _make_sc_kernel(B, L, packed64.shape[0], H, C=16)
    return sc(x_flat, packed64, utab)


# speedup vs baseline: 4.6282x; 1.0046x over previous
import functools

import jax
import jax.numpy as jnp
from jax import lax
from jax.experimental import pallas as pl
from jax.experimental.pallas import tpu as pltpu
from jax.experimental.pallas import tpu_sc as plsc

LANES = 16


TC_BN = 16384


def _utable_tc(tableT, W, b):
    V = tableT.shape[1]
    H = tableT.shape[0]
    BN = TC_BN
    grid = (V + BN - 1) // BN
    R = grid * (BN // 2)

    def body(t_ref, w_ref, b_ref, u_ref, p_ref):
        blk = t_ref[...]
        wb = jnp.broadcast_to(w_ref[...], (H, BN))
        s = jnp.sum(blk * wb, axis=0) + b_ref[0]
        u_ref[...] = jnp.exp(jnp.tanh(s))
        ta = jnp.swapaxes(blk[:, : BN // 2], 0, 1)
        tb = jnp.swapaxes(blk[:, BN // 2:], 0, 1)
        p_ref[:, pl.ds(0, H)] = ta
        p_ref[:, pl.ds(H, H)] = tb

    return pl.pallas_call(
        body,
        grid=grid,
        in_specs=[
            pl.BlockSpec((H, BN), lambda j: (0, j)),
            pl.BlockSpec((H, 1), lambda j: (0, 0)),
            pl.BlockSpec(memory_space=pltpu.SMEM),
        ],
        out_specs=[
            pl.BlockSpec((BN,), lambda j: (j,)),
            pl.BlockSpec((BN // 2, 2 * H), lambda j: (j, 0)),
        ],
        out_shape=[
            jax.ShapeDtypeStruct((V,), jnp.float32),
            jax.ShapeDtypeStruct((R, 2 * H), jnp.float32),
        ],
    )(tableT, W, b)


def _make_sc_kernel(B, L, R64, H, C):
    info = plsc.get_sparse_core_info()
    NC, NS = info.num_cores, info.num_subcores
    NW = NC * NS
    samples_per_w = B // NW
    n_chunks = samples_per_w // C
    toks = C * L
    toks2 = 2 * toks
    HV = H // LANES
    n_ugrp = (L + LANES - 1) // LANES
    n_grp2 = toks2 // LANES
    KB = TC_BN // 2
    assert n_chunks % 2 == 0 and toks % LANES == 0

    mesh = plsc.VectorSubcoreMesh(core_axis_name="c", subcore_axis_name="s")

    @functools.partial(
        pl.kernel,
        mesh=mesh,
        out_type=jax.ShapeDtypeStruct((B, H), jnp.float32),
        compiler_params=pltpu.CompilerParams(
            use_tc_tiling_on_sc=False, needs_layout_passes=False
        ),
        scratch_types=[
            pltpu.VMEM((toks2,), jnp.int32),
            pltpu.VMEM((toks2,), jnp.int32),
            pltpu.VMEM((toks, H), jnp.float32),
            pltpu.VMEM((toks, H), jnp.float32),
            pltpu.VMEM((toks,), jnp.float32),
            pltpu.VMEM((toks,), jnp.float32),
            pltpu.VMEM((C, H), jnp.float32),
            pltpu.SemaphoreType.DMA,
            pltpu.SemaphoreType.DMA,
            pltpu.SemaphoreType.DMA,
            pltpu.SemaphoreType.DMA,
        ],
    )
    def k(x_ref, table_ref, utab_ref, out_ref,
          idx_v, idx2_v, emb_a, emb_b, u_a, u_b, out_v,
          sem_ra, sem_rb, sem_ua, sem_ub):
        cid = lax.axis_index("c")
        sid = lax.axis_index("s")
        wid = sid * NC + cid
        sample0 = wid * samples_per_w

        def compute(emb_v, u_v, chunk):
            def sample_body(s, _):
                row0 = s * L
                bases = [min(g * LANES, L - LANES) for g in range(n_ugrp)]
                uvecs = [u_v[pl.ds(row0 + bg, LANES)] for bg in bases]
                acc = [jnp.zeros((LANES,), jnp.float32) for _ in range(HV)]
                dacc = jnp.zeros((LANES,), jnp.float32)
                for l in range(L):
                    g = min(l // LANES, n_ugrp - 1)
                    lane = l - bases[g]
                    u = jnp.broadcast_to(uvecs[g][lane], (LANES,))
                    for j in range(HV):
                        acc[j] = acc[j] + u * emb_v[row0 + l,
                                                    pl.ds(j * LANES, LANES)]
                    dacc = dacc + u
                inv = 1.0 / dacc
                for j in range(HV):
                    out_v[s, pl.ds(j * LANES, LANES)] = acc[j] * inv
                return _

            lax.fori_loop(0, C, sample_body, 0)
            pltpu.sync_copy(out_v, out_ref.at[pl.ds(sample0 + chunk * C, C)])

        def pair_body(i, _):
            c0 = i * 2
            c1 = c0 + 1
            pltpu.sync_copy(
                x_ref.at[pl.ds((sample0 + c0 * C) * L, toks2)], idx_v)
            for g in range(n_grp2):
                sl = pl.ds(g * LANES, LANES)
                v = idx_v[sl]
                hi = lax.shift_left(lax.shift_right_logical(v, 15), 15)
                mid = lax.shift_left(jnp.bitwise_and(v, KB - 1), 1)
                par = jnp.bitwise_and(lax.shift_right_logical(v, 14), 1)
                idx2_v[sl] = jnp.bitwise_or(hi, jnp.bitwise_or(mid, par))
            h_ra = pltpu.async_copy(
                table_ref.at[idx2_v.at[pl.ds(0, toks)]], emb_a, sem_ra)
            h_ua = pltpu.async_copy(
                utab_ref.at[idx_v.at[pl.ds(0, toks)]], u_a, sem_ua)
            h_rb = pltpu.async_copy(
                table_ref.at[idx2_v.at[pl.ds(toks, toks)]], emb_b, sem_rb)
            h_ub = pltpu.async_copy(
                utab_ref.at[idx_v.at[pl.ds(toks, toks)]], u_b, sem_ub)
            h_ra.wait()
            h_ua.wait()
            compute(emb_a, u_a, c0)
            h_rb.wait()
            h_ub.wait()
            compute(emb_b, u_b, c1)
            return _

        lax.fori_loop(0, n_chunks // 2, pair_body, 0)

    return k


def kernel(x, table, W, b):
    B, L = x.shape
    V, H = table.shape
    x_flat = x.reshape(B * L)
    utab, packed = _utable_tc(table.T, W, b)
    packed64 = packed.reshape(packed.shape[0] * 2, H)
    sc = _make_sc_kernel(B, L, packed64.shape[0], H, C=16)
    return sc(x_flat, packed64, utab)


# speedup vs baseline: 4.6369x; 1.0019x over previous
import functools

import jax
import jax.numpy as jnp
from jax import lax
from jax.experimental import pallas as pl
from jax.experimental.pallas import tpu as pltpu
from jax.experimental.pallas import tpu_sc as plsc

LANES = 16


TC_BN = 16384


def _utable_tc(tableT, W, b):
    V = tableT.shape[1]
    H = tableT.shape[0]
    BN = TC_BN
    grid = (V + BN - 1) // BN
    R = grid * (BN // 2)

    def body(t_ref, w_ref, b_ref, u_ref, p_ref):
        blk = t_ref[...]
        wb = jnp.broadcast_to(w_ref[...], (H, BN))
        s = jnp.sum(blk * wb, axis=0) + b_ref[0]
        u_ref[...] = jnp.exp(jnp.tanh(s))
        ta = jnp.swapaxes(blk[:, : BN // 2], 0, 1)
        tb = jnp.swapaxes(blk[:, BN // 2:], 0, 1)
        p_ref[:, pl.ds(0, H)] = ta
        p_ref[:, pl.ds(H, H)] = tb

    return pl.pallas_call(
        body,
        grid=grid,
        in_specs=[
            pl.BlockSpec((H, BN), lambda j: (0, j)),
            pl.BlockSpec((H, 1), lambda j: (0, 0)),
            pl.BlockSpec(memory_space=pltpu.SMEM),
        ],
        out_specs=[
            pl.BlockSpec((BN,), lambda j: (j,)),
            pl.BlockSpec((BN // 2, 2 * H), lambda j: (j, 0)),
        ],
        out_shape=[
            jax.ShapeDtypeStruct((V,), jnp.float32),
            jax.ShapeDtypeStruct((R, 2 * H), jnp.float32),
        ],
    )(tableT, W, b)


def _make_sc_kernel(B, L, R64, H, C):
    info = plsc.get_sparse_core_info()
    NC, NS = info.num_cores, info.num_subcores
    NW = NC * NS
    samples_per_w = B // NW
    n_chunks = samples_per_w // C
    toks = C * L
    toks2 = 2 * toks
    HV = H // LANES
    n_ugrp = (L + LANES - 1) // LANES
    n_grp2 = toks2 // LANES
    KB = TC_BN // 2
    LB = TC_BN.bit_length() - 1
    assert n_chunks % 2 == 0 and toks % LANES == 0

    mesh = plsc.VectorSubcoreMesh(core_axis_name="c", subcore_axis_name="s")

    @functools.partial(
        pl.kernel,
        mesh=mesh,
        out_type=jax.ShapeDtypeStruct((B, H), jnp.float32),
        compiler_params=pltpu.CompilerParams(
            use_tc_tiling_on_sc=False, needs_layout_passes=False
        ),
        scratch_types=[
            pltpu.VMEM((toks2,), jnp.int32),
            pltpu.VMEM((toks2,), jnp.int32),
            pltpu.VMEM((toks, H), jnp.float32),
            pltpu.VMEM((toks, H), jnp.float32),
            pltpu.VMEM((toks,), jnp.float32),
            pltpu.VMEM((toks,), jnp.float32),
            pltpu.VMEM((C, H), jnp.float32),
            pltpu.SemaphoreType.DMA,
            pltpu.SemaphoreType.DMA,
            pltpu.SemaphoreType.DMA,
            pltpu.SemaphoreType.DMA,
        ],
    )
    def k(x_ref, table_ref, utab_ref, out_ref,
          idx_v, idx2_v, emb_a, emb_b, u_a, u_b, out_v,
          sem_ra, sem_rb, sem_ua, sem_ub):
        cid = lax.axis_index("c")
        sid = lax.axis_index("s")
        wid = sid * NC + cid
        sample0 = wid * samples_per_w

        def compute(emb_v, u_v, chunk):
            def sample_body(s, _):
                row0 = s * L
                bases = [min(g * LANES, L - LANES) for g in range(n_ugrp)]
                uvecs = [u_v[pl.ds(row0 + bg, LANES)] for bg in bases]
                acc = [jnp.zeros((LANES,), jnp.float32) for _ in range(HV)]
                dacc = jnp.zeros((LANES,), jnp.float32)
                for l in range(L):
                    g = min(l // LANES, n_ugrp - 1)
                    lane = l - bases[g]
                    u = jnp.broadcast_to(uvecs[g][lane], (LANES,))
                    for j in range(HV):
                        acc[j] = acc[j] + u * emb_v[row0 + l,
                                                    pl.ds(j * LANES, LANES)]
                    dacc = dacc + u
                inv = 1.0 / dacc
                for j in range(HV):
                    out_v[s, pl.ds(j * LANES, LANES)] = acc[j] * inv
                return _

            lax.fori_loop(0, C, sample_body, 0)
            pltpu.sync_copy(out_v, out_ref.at[pl.ds(sample0 + chunk * C, C)])

        def pair_body(i, _):
            c0 = i * 2
            c1 = c0 + 1
            pltpu.sync_copy(
                x_ref.at[pl.ds((sample0 + c0 * C) * L, toks2)], idx_v)
            for g in range(n_grp2):
                sl = pl.ds(g * LANES, LANES)
                v = idx_v[sl]
                hi = lax.shift_left(lax.shift_right_logical(v, LB), LB)
                mid = lax.shift_left(jnp.bitwise_and(v, KB - 1), 1)
                par = jnp.bitwise_and(lax.shift_right_logical(v, LB - 1), 1)
                idx2_v[sl] = jnp.bitwise_or(hi, jnp.bitwise_or(mid, par))
            h_ra = pltpu.async_copy(
                table_ref.at[idx2_v.at[pl.ds(0, toks)]], emb_a, sem_ra)
            h_ua = pltpu.async_copy(
                utab_ref.at[idx_v.at[pl.ds(0, toks)]], u_a, sem_ua)
            h_rb = pltpu.async_copy(
                table_ref.at[idx2_v.at[pl.ds(toks, toks)]], emb_b, sem_rb)
            h_ub = pltpu.async_copy(
                utab_ref.at[idx_v.at[pl.ds(toks, toks)]], u_b, sem_ub)
            h_ra.wait()
            h_ua.wait()
            compute(emb_a, u_a, c0)
            h_rb.wait()
            h_ub.wait()
            compute(emb_b, u_b, c1)
            return _

        lax.fori_loop(0, n_chunks // 2, pair_body, 0)

    return k


def kernel(x, table, W, b):
    B, L = x.shape
    V, H = table.shape
    x_flat = x.reshape(B * L)
    utab, packed = _utable_tc(table.T, W, b)
    packed64 = packed.reshape(packed.shape[0] * 2, H)
    sc = _make_sc_kernel(B, L, packed64.shape[0], H, C=16)
    return sc(x_flat, packed64, utab)
